# Initial kernel scaffold; baseline (speedup 1.0000x reference)
import jax
import jax.numpy as jnp
from jax.experimental import pallas as pl


def kernel(edge_index, x, edge_weight):
    raise NotImplementedError("write your pallas kernel here")


# speedup vs baseline: 7.6184x; 7.6184x over previous
import functools

import jax
import jax.numpy as jnp
from jax import lax
from jax.experimental import pallas as pl
from jax.experimental.pallas import tpu as pltpu
from jax.experimental.pallas import tpu_sc as plsc

N = 10000
N_PAD = 10112
E = 320000
D = 128
NC = 2
NS = 16
NW = NC * NS
CHUNK = 128
RPT = 80
E_PAD = NW * RPT * CHUNK
SLAB = N_PAD // NS


def _sc_kernel(x_hbm, src_hbm, dst_hbm, w_hbm, part_hbm,
               src_v, dst_v, w_v, rows, acc, sem):
    c = lax.axis_index("c")
    s = lax.axis_index("s")
    wid = c * NS + s
    base = wid * RPT

    pltpu.sync_copy(src_hbm.at[pl.ds(base, RPT)], src_v)
    pltpu.sync_copy(dst_hbm.at[pl.ds(base, RPT)], dst_v)
    pltpu.sync_copy(w_hbm.at[pl.ds(base * CHUNK, RPT * CHUNK)], w_v)

    zero = jnp.zeros((16,), jnp.float32)

    def zbody(k, _):
        for cc in range(8):
            rows[k, pl.ds(cc * 16, 16)] = zero
        return 0

    lax.fori_loop(0, CHUNK, zbody, 0)
    for t in range(4):
        pltpu.sync_copy(rows, acc.at[pl.ds(s * SLAB + t * CHUNK, CHUNK)])
    pltpu.sync_copy(rows.at[pl.ds(0, SLAB - 4 * CHUNK)],
                    acc.at[pl.ds(s * SLAB + 4 * CHUNK, SLAB - 4 * CHUNK)])
    plsc.subcore_barrier()

    def body(j, _):
        pltpu.async_copy(x_hbm.at[src_v.at[j]], rows, sem).wait()

        def mul_body(g, _):
            w16 = w_v[pl.ds(j * CHUNK + g * 16, 16)]
            for kk in range(16):
                wv = jnp.broadcast_to(w16[kk], (16,))
                row = g * 16 + kk
                for cc in range(8):
                    sl = pl.ds(cc * 16, 16)
                    rows[row, sl] = rows[row, sl] * wv
            return 0

        lax.fori_loop(0, CHUNK // 16, mul_body, 0)

        pltpu.sync_copy(rows, acc.at[dst_v.at[j]], add=True)
        return 0

    lax.fori_loop(0, RPT, body, 0)
    plsc.subcore_barrier()

    pltpu.sync_copy(acc.at[pl.ds(s * SLAB, SLAB)],
                    part_hbm.at[c, pl.ds(s * SLAB, SLAB)])


def _combine_body(p_ref, o_ref):
    o_ref[...] = p_ref[0] + p_ref[1]


def kernel(edge_index, x, edge_weight):
    pad = E_PAD - E
    pad_idx = (jnp.arange(pad, dtype=jnp.int32) % N)
    src = jnp.concatenate([edge_index[0].astype(jnp.int32), pad_idx])
    dst = jnp.concatenate([edge_index[1].astype(jnp.int32), pad_idx])
    w = jnp.concatenate([edge_weight, jnp.zeros((pad,), jnp.float32)])
    rtot = E_PAD // CHUNK
    src2 = src.reshape(rtot, CHUNK)
    dst2 = dst.reshape(rtot, CHUNK)

    mesh = plsc.VectorSubcoreMesh(core_axis_name="c", subcore_axis_name="s",
                                  num_cores=NC, num_subcores=NS)
    part = pl.kernel(
        _sc_kernel,
        out_type=jax.ShapeDtypeStruct((NC, N_PAD, D), jnp.float32),
        mesh=mesh,
        scratch_types=[
            pltpu.VMEM((RPT, CHUNK), jnp.int32),
            pltpu.VMEM((RPT, CHUNK), jnp.int32),
            pltpu.VMEM((RPT * CHUNK,), jnp.float32),
            pltpu.VMEM((CHUNK, D), jnp.float32),
            pltpu.VMEM_SHARED((N_PAD, D), jnp.float32),
            pltpu.SemaphoreType.DMA,
        ],
    )(x, src2, dst2, w)

    out = pl.pallas_call(
        _combine_body,
        grid=(10,),
        in_specs=[pl.BlockSpec((NC, N // 10, D), lambda i: (0, i, 0))],
        out_specs=pl.BlockSpec((N // 10, D), lambda i: (i, 0)),
        out_shape=jax.ShapeDtypeStruct((N, D), jnp.float32),
    )(part)
    return out
